# baseline (speedup 1.0000x reference)
import functools

import jax
import jax.numpy as jnp
from jax import lax
from jax.experimental import pallas as pl
from jax.experimental.pallas import tpu as pltpu
from jax.experimental.pallas import tpu_sc as plsc

N_NODES = 10000
N_EDGES = 320000
N_HALF = N_EDGES // 2
N_GRAPHS = 256
E_BLK = 3200

_NC = 2
_NS = 16
_NW = _NC * _NS
_ANR = 10240
_APS = _ANR // _NS


def _leaky(x):
    return jnp.where(x >= 0, x, 0.2 * x)


def _mesh():
    return plsc.VectorSubcoreMesh(core_axis_name="c", subcore_axis_name="s")


def _make_gather(n_rows, gc):
    epw = n_rows // _NW
    nch = epw // gc

    def body(nf_hbm, idx1_hbm, idx2_hbm, n1_hbm, n2_hbm,
             idx1_v, idx2_v, rows0, rows1, g0, g1, wb0, wb1):
        cid = lax.axis_index("c")
        sid = lax.axis_index("s")
        wid = sid * _NC + cid
        base = wid * epw
        pltpu.sync_copy(idx1_hbm.at[pl.ds(base, epw)], idx1_v)
        pltpu.sync_copy(idx2_hbm.at[pl.ds(base, epw)], idx2_v)

        def chunk(c, carry):
            off = base + c * gc
            poff = base + (c - 1) * gc

            @pl.when(c > 0)
            def _drain0():
                pltpu.make_async_copy(
                    rows0, n1_hbm.at[pl.ds(poff, gc)], wb0).wait()
            h1 = pltpu.async_copy(
                nf_hbm.at[idx1_v.at[pl.ds(c * gc, gc)]], rows0, g0)

            @pl.when(c > 0)
            def _drain1():
                pltpu.make_async_copy(
                    rows1, n2_hbm.at[pl.ds(poff, gc)], wb1).wait()
            h2 = pltpu.async_copy(
                nf_hbm.at[idx2_v.at[pl.ds(c * gc, gc)]], rows1, g1)

            h1.wait()
            pltpu.async_copy(rows0, n1_hbm.at[pl.ds(off, gc)], wb0)
            h2.wait()
            pltpu.async_copy(rows1, n2_hbm.at[pl.ds(off, gc)], wb1)
            return carry

        lax.fori_loop(0, nch, chunk, 0)
        loff = base + (nch - 1) * gc
        pltpu.make_async_copy(rows0, n1_hbm.at[pl.ds(loff, gc)], wb0).wait()
        pltpu.make_async_copy(rows1, n2_hbm.at[pl.ds(loff, gc)], wb1).wait()

    def call(nf, idx1, idx2):
        k = pl.kernel(
            body,
            out_type=[jax.ShapeDtypeStruct((n_rows, 128), jnp.float32)] * 2,
            mesh=_mesh(),
            scratch_types=[
                pltpu.VMEM((epw,), jnp.int32),
                pltpu.VMEM((epw,), jnp.int32),
                pltpu.VMEM((gc, 128), jnp.float32),
                pltpu.VMEM((gc, 128), jnp.float32),
                pltpu.SemaphoreType.DMA,
                pltpu.SemaphoreType.DMA,
                pltpu.SemaphoreType.DMA,
                pltpu.SemaphoreType.DMA,
            ],
        )
        return k(nf, idx1, idx2)

    return call


_gather_half = _make_gather(N_HALF, 200)


def _make_scatter(n_rows, gcs, count_mode=False):
    eps = n_rows // _NS
    nch = eps // gcs
    npair = nch // 2

    def fetch(v_hbm, idx_hbm, off, idx_b, row_b, sem):
        pltpu.async_copy(idx_hbm.at[pl.ds(off, gcs)], idx_b, sem)
        pltpu.async_copy(v_hbm.at[pl.ds(off, gcs)], row_b, sem)

    def await_fetch(v_hbm, idx_hbm, off, idx_b, row_b, sem):
        pltpu.make_async_copy(idx_hbm.at[pl.ds(off, gcs)], idx_b, sem).wait()
        pltpu.make_async_copy(v_hbm.at[pl.ds(off, gcs)], row_b, sem).wait()

    def value_loop(v_hbm, idx_hbm, sid, acc,
                   idx_a, idx_b, row_a, row_b, sem_a, sem_b):
        base = sid * eps
        fetch(v_hbm, idx_hbm, base, idx_a, row_a, sem_a)

        def pair(i, carry):
            off_a = base + 2 * i * gcs
            off_b = off_a + gcs
            fetch(v_hbm, idx_hbm, off_b, idx_b, row_b, sem_b)
            await_fetch(v_hbm, idx_hbm, off_a, idx_a, row_a, sem_a)
            pltpu.sync_copy(row_a, acc.at[idx_a], add=True)

            @pl.when(i < npair - 1)
            def _next_a():
                fetch(v_hbm, idx_hbm, off_b + gcs, idx_a, row_a, sem_a)
            await_fetch(v_hbm, idx_hbm, off_b, idx_b, row_b, sem_b)
            pltpu.sync_copy(row_b, acc.at[idx_b], add=True)
            return carry

        lax.fori_loop(0, npair, pair, 0)
        if nch % 2 == 1:
            off_l = base + (nch - 1) * gcs
            fetch(v_hbm, idx_hbm, off_l, idx_a, row_a, sem_a)
            await_fetch(v_hbm, idx_hbm, off_l, idx_a, row_a, sem_a)
            pltpu.sync_copy(row_a, acc.at[idx_a], add=True)

    def count_loop(idx_hbm, sid, acc, idx_a, idx_b, row_a, sem_a, sem_b):
        base = sid * eps
        pltpu.async_copy(idx_hbm.at[pl.ds(base, gcs)], idx_a, sem_a)

        def chunk(c, carry):
            off = base + c * gcs
            noff = off + gcs

            @pl.when((c % 2 == 0) & (c < nch - 1))
            def _pf_b():
                pltpu.async_copy(idx_hbm.at[pl.ds(noff, gcs)], idx_b, sem_b)

            @pl.when(c % 2 == 0)
            def _even():
                pltpu.make_async_copy(
                    idx_hbm.at[pl.ds(off, gcs)], idx_a, sem_a).wait()
                pltpu.sync_copy(row_a, acc.at[idx_a], add=True)

            @pl.when((c % 2 == 1) & (c < nch - 1))
            def _pf_a():
                pltpu.async_copy(idx_hbm.at[pl.ds(noff, gcs)], idx_a, sem_a)

            @pl.when(c % 2 == 1)
            def _odd():
                pltpu.make_async_copy(
                    idx_hbm.at[pl.ds(off, gcs)], idx_b, sem_b).wait()
                pltpu.sync_copy(row_a, acc.at[idx_b], add=True)
            return carry
        lax.fori_loop(0, nch, chunk, 0)

    def body(v1_hbm, v2_hbm, idx_hbm, zz_hbm, out_hbm,
             idx_a, idx_b, row_a, row_b, sem_a, sem_b, acc):
        cid = lax.axis_index("c")
        sid = lax.axis_index("s")
        pltpu.sync_copy(zz_hbm, acc.at[pl.ds(sid * _APS, _APS)])
        plsc.subcore_barrier()

        @pl.when(cid == 0)
        def _core0():
            value_loop(v1_hbm, idx_hbm, sid, acc,
                       idx_a, idx_b, row_a, row_b, sem_a, sem_b)

        @pl.when(cid == 1)
        def _core1():
            if count_mode:
                pltpu.sync_copy(v2_hbm, row_a)
                count_loop(idx_hbm, sid, acc, idx_a, idx_b, row_a,
                           sem_a, sem_b)
            else:
                value_loop(v2_hbm, idx_hbm, sid, acc,
                           idx_a, idx_b, row_a, row_b, sem_a, sem_b)

        plsc.subcore_barrier()
        pltpu.sync_copy(acc.at[pl.ds(sid * _APS, _APS)],
                        out_hbm.at[pl.ds(cid * _ANR + sid * _APS, _APS)])

    def call(v1, v2, idx1):
        zz = jnp.zeros((_APS, 128), jnp.float32)
        k = pl.kernel(
            body,
            out_type=jax.ShapeDtypeStruct((2 * _ANR, 128), jnp.float32),
            mesh=_mesh(),
            scratch_types=[
                pltpu.VMEM((gcs,), jnp.int32),
                pltpu.VMEM((gcs,), jnp.int32),
                pltpu.VMEM((gcs, 128), jnp.float32),
                pltpu.VMEM((gcs, 128), jnp.float32),
                pltpu.SemaphoreType.DMA,
                pltpu.SemaphoreType.DMA,
                pltpu.VMEM_SHARED((_ANR, 128), jnp.float32),
            ],
        )
        return k(v1, v2, idx1, zz)

    return call


_scatter_half = _make_scatter(N_HALF, 80)
_sum_count_half = _make_scatter(N_HALF, 80, count_mode=True)


def _p1_body(n1_ref, n2_ref, ef_ref, wa_ref, wb_ref, wc_ref, b0_ref,
             w1_ref, b1_ref, w2_ref, b2_ref, h3_ref, stats_ref):
    i = pl.program_id(0)
    h = (jnp.dot(n1_ref[...], wa_ref[...], preferred_element_type=jnp.float32)
         + jnp.dot(n2_ref[...], wb_ref[...], preferred_element_type=jnp.float32)
         + jnp.dot(ef_ref[...], wc_ref[...], preferred_element_type=jnp.float32)
         + b0_ref[...])
    h = _leaky(h)
    h = _leaky(jnp.dot(h, w1_ref[...], preferred_element_type=jnp.float32)
               + b1_ref[...])
    h3 = jnp.dot(h, w2_ref[...], preferred_element_type=jnp.float32) + b2_ref[...]
    h3_ref[...] = h3

    @pl.when(i == 0)
    def _init():
        stats_ref[...] = jnp.zeros_like(stats_ref)

    s1 = jnp.sum(h3, axis=0, keepdims=True)
    s2 = jnp.sum(h3 * h3, axis=0, keepdims=True)
    stats_ref[...] += jnp.concatenate(
        [s1, s2, jnp.zeros((6, h3.shape[1]), jnp.float32)], axis=0)


def _edge_pass1(n1, n2, ef, wa, wb, wc, b0, w1, b1, w2, b2):
    nrows = n1.shape[0]
    eb = lambda i: (i, 0)
    fb = lambda i: (0, 0)
    espec = pl.BlockSpec((E_BLK, 128), eb)
    wspec = pl.BlockSpec((128, 128), fb)
    vspec = pl.BlockSpec((1, 128), fb)
    return pl.pallas_call(
        _p1_body,
        grid=(nrows // E_BLK,),
        in_specs=[espec, espec, espec, wspec, wspec, wspec, vspec,
                  wspec, vspec, wspec, vspec],
        out_specs=[espec, pl.BlockSpec((8, 128), fb)],
        out_shape=[
            jax.ShapeDtypeStruct((nrows, 128), jnp.float32),
            jax.ShapeDtypeStruct((8, 128), jnp.float32),
        ],
    )(n1, n2, ef, wa, wb, wc, b0, w1, b1, w2, b2)


def _p2_body(h3_ref, n1_ref, n2_ref, ef_ref, scale_ref, shift_ref,
             wsa_ref, wsb_ref, wsc_ref, bs0_ref, ws1_ref, bs1_ref,
             wma_ref, wmb_ref, wmc_ref, bm0_ref, wm1_ref, bm1_ref,
             enew_ref, v1_ref, v2_ref):
    ek = h3_ref[...] * scale_ref[...] + shift_ref[...]
    enew_ref[...] = ef_ref[...] + ek
    n1 = n1_ref[...]
    n2 = n2_ref[...]
    sh = _leaky(
        jnp.dot(n1, wsa_ref[...], preferred_element_type=jnp.float32)
        + jnp.dot(n2, wsb_ref[...], preferred_element_type=jnp.float32)
        + jnp.dot(ek, wsc_ref[...], preferred_element_type=jnp.float32)
        + bs0_ref[...])
    sij = jnp.dot(sh, ws1_ref[...], preferred_element_type=jnp.float32) + bs1_ref[...]
    w = jnp.exp(jnp.minimum(sij, 60.0))
    mh = _leaky(
        jnp.dot(n1, wma_ref[...], preferred_element_type=jnp.float32)
        + jnp.dot(n2, wmb_ref[...], preferred_element_type=jnp.float32)
        + jnp.dot(ek, wmc_ref[...], preferred_element_type=jnp.float32)
        + bm0_ref[...])
    mij = jnp.dot(mh, wm1_ref[...],
                  preferred_element_type=jnp.float32) + bm1_ref[...]
    v1_ref[...] = w * mij
    v2_ref[...] = w


def _edge_pass2(h3, n1, n2, ef, scale, shift, wsa, wsb, wsc, bs0, ws1, bs1,
                wma, wmb, wmc, bm0, wm1, bm1):
    nrows = n1.shape[0]
    eb = lambda i: (i, 0)
    fb = lambda i: (0, 0)
    espec = pl.BlockSpec((E_BLK, 128), eb)
    wspec = pl.BlockSpec((128, 128), fb)
    vspec = pl.BlockSpec((1, 128), fb)
    return pl.pallas_call(
        _p2_body,
        grid=(nrows // E_BLK,),
        in_specs=[espec, espec, espec, espec, vspec, vspec,
                  wspec, wspec, wspec, vspec, wspec, vspec,
                  wspec, wspec, wspec, vspec, wspec, vspec],
        out_specs=[espec, espec, espec],
        out_shape=[jax.ShapeDtypeStruct((nrows, 128), jnp.float32)] * 3,
    )(h3, n1, n2, ef, scale, shift, wsa, wsb, wsc, bs0, ws1, bs1,
      wma, wmb, wmc, bm0, wm1, bm1)


def _bn_from_stats(stats, n, g, b, eps=1e-5):
    mean = stats[0] / n
    var = stats[1] / n - mean * mean
    scale = g / jnp.sqrt(var + eps)
    shift = b - mean * scale
    return scale[None, :], shift[None, :]


def _attn_layer(layer, nf, ixs, efa, efb):
    i1a, i1b, i2a, i2b = ixs
    w_phi0 = layer["phi_e"][0]["w"]
    w_a0 = layer["fcnna"][0]["w"]
    w_m0 = layer["fcnnm"][0]["w"]

    p1w = (w_phi0[:128], w_phi0[128:256], w_phi0[256:384],
           layer["phi_e"][0]["b"][None, :],
           layer["phi_e"][1]["w"], layer["phi_e"][1]["b"][None, :],
           layer["phi_e"][2]["w"], layer["phi_e"][2]["b"][None, :])
    p2w = (w_a0[:128], w_a0[128:256], w_a0[256:384],
           layer["fcnna"][0]["b"][None, :],
           layer["fcnna"][1]["w"], layer["fcnna"][1]["b"][None, :],
           w_m0[:128], w_m0[128:256], w_m0[256:384],
           layer["fcnnm"][0]["b"][None, :],
           layer["fcnnm"][1]["w"], layer["fcnnm"][1]["b"][None, :])

    n1a, n2a = _gather_half(nf, i1a, i2a)
    n1b, n2b = _gather_half(nf, i1b, i2b)

    h3a, sta = _edge_pass1(n1a, n2a, efa, *p1w)
    h3b, stb = _edge_pass1(n1b, n2b, efb, *p1w)

    scale, shift = _bn_from_stats(sta + stb, float(N_EDGES),
                                  layer["bn1_g"], layer["bn1_b"])

    enewa, v1a, v2a = _edge_pass2(h3a, n1a, n2a, efa, scale, shift, *p2w)
    sega = _scatter_half(v1a, v2a, i1a)
    enewb, v1b, v2b = _edge_pass2(h3b, n1b, n2b, efb, scale, shift, *p2w)
    segb = _scatter_half(v1b, v2b, i1b)

    seg = sega + segb
    msg = seg[:N_NODES] / (seg[_ANR:_ANR + N_NODES] + 1e-16)

    mu = jnp.mean(msg, axis=0)
    var = jnp.var(msg, axis=0)
    nsc = layer["bn2_g"] / jnp.sqrt(var + 1e-5)
    node_new = nf + (msg - mu) * nsc + layer["bn2_b"]
    return node_new, enewa, enewb


def kernel(node_fea, edge_fea, idx1, idx2, idx3, params):
    nf = params["v_emb"][node_fea]
    ef = edge_fea @ params["e_emb"]["w"] + params["e_emb"]["b"]
    efa, efb = ef[:N_HALF], ef[N_HALF:]
    ixs = (idx1[:N_HALF], idx1[N_HALF:], idx2[:N_HALF], idx2[N_HALF:])
    for layer in params["attns"]:
        nf, efa, efb = _attn_layer(layer, nf, ixs, efa, efb)

    ones = jnp.ones((80, 128), jnp.float32)
    pa = _sum_count_half(efa, ones, ixs[0])
    pb = _sum_count_half(efb, ones, ixs[1])
    pooled = pa + pb
    cnt = pooled[_ANR:_ANR + N_NODES, 0]
    vi_e_bar = pooled[:N_NODES] / jnp.maximum(cnt, 1.0)[:, None]
    crys = jnp.concatenate([vi_e_bar, nf], axis=1)
    cnt3 = jax.ops.segment_sum(jnp.ones((N_NODES,), jnp.float32), idx3,
                               num_segments=N_GRAPHS)
    crys = jax.ops.segment_sum(crys, idx3, num_segments=N_GRAPHS)
    crys = crys / jnp.maximum(cnt3, 1.0)[:, None]
    h = _leaky(crys @ params["conv_to_fc"]["w"] + params["conv_to_fc"]["b"])
    for fc in params["fcs"]:
        h = _leaky(h @ fc["w"] + fc["b"])
    return h @ params["fc_out"]["w"] + params["fc_out"]["b"]


# speedup vs baseline: 1.1000x; 1.1000x over previous
import functools

import jax
import jax.numpy as jnp
from jax import lax
from jax.experimental import pallas as pl
from jax.experimental.pallas import tpu as pltpu
from jax.experimental.pallas import tpu_sc as plsc

N_NODES = 10000
N_EDGES = 320000
N_HALF = N_EDGES // 2
N_GRAPHS = 256
E_BLK = 3200

_NC = 2
_NS = 16
_NW = _NC * _NS
_ANR = 10240
_APS = _ANR // _NS


def _leaky(x):
    return jnp.where(x >= 0, x, 0.2 * x)


def _mesh():
    return plsc.VectorSubcoreMesh(core_axis_name="c", subcore_axis_name="s")


def _make_gather(n_rows, gc, nb=4):
    epw = n_rows // _NW
    nch = epw // gc
    ngrp = (nch - 1) // nb

    def body(nf_hbm, idx1_hbm, idx2_hbm, n1_hbm, n2_hbm,
             idx1_v, idx2_v, b0, b1, b2, b3,
             g0, g1, g2, g3, w0, w1, w2, w3):
        bufs = (b0, b1, b2, b3)
        gs = (g0, g1, g2, g3)
        ws = (w0, w1, w2, w3)
        cid = lax.axis_index("c")
        sid = lax.axis_index("s")
        wid = sid * _NC + cid
        base = wid * epw
        pltpu.sync_copy(idx1_hbm.at[pl.ds(base, epw)], idx1_v)
        pltpu.sync_copy(idx2_hbm.at[pl.ds(base, epw)], idx2_v)

        def phase(idx_v, out_hbm):
            def start_g(b, c):
                pltpu.async_copy(
                    nf_hbm.at[idx_v.at[pl.ds(c * gc, gc)]], bufs[b], gs[b])

            def wait_g(b, c):
                pltpu.make_async_copy(
                    nf_hbm.at[idx_v.at[pl.ds(c * gc, gc)]],
                    bufs[b], gs[b]).wait()

            def start_w(b, c):
                pltpu.async_copy(
                    bufs[b], out_hbm.at[pl.ds(base + c * gc, gc)], ws[b])

            def wait_w(b, c):
                pltpu.make_async_copy(
                    bufs[b], out_hbm.at[pl.ds(base + c * gc, gc)],
                    ws[b]).wait()

            start_g(0, 0)
            start_g(1, 1)

            def group(i, carry):
                c0 = i * nb
                for j in range(nb):
                    c = c0 + j
                    bn = (j + 2) % nb

                    @pl.when(c + 2 < nch)
                    def _pf():
                        @pl.when(c >= 2)
                        def _dr():
                            wait_w(bn, c - 2)
                        start_g(bn, c + 2)
                    wait_g(j, c)
                    start_w(j, c)
                return carry

            lax.fori_loop(0, ngrp, group, 0)
            for c in range(ngrp * nb, nch):
                wait_g(c % nb, c)
                start_w(c % nb, c)
            for c in range(nch - nb, nch):
                wait_w(c % nb, c)

        phase(idx1_v, n1_hbm)
        phase(idx2_v, n2_hbm)

    def call(nf, idx1, idx2):
        k = pl.kernel(
            body,
            out_type=[jax.ShapeDtypeStruct((n_rows, 128), jnp.float32)] * 2,
            mesh=_mesh(),
            scratch_types=[
                pltpu.VMEM((epw,), jnp.int32),
                pltpu.VMEM((epw,), jnp.int32),
            ] + [pltpu.VMEM((gc, 128), jnp.float32)] * 4
            + [pltpu.SemaphoreType.DMA] * 8,
        )
        return k(nf, idx1, idx2)

    return call


_gather_half = _make_gather(N_HALF, 200)


_PAD_ROW = 10200


def _make_scatter(n_rows, gcs, count_mode=False):
    eps = n_rows // _NS
    nch = eps // gcs
    npair = nch // 2
    tail = eps - nch * gcs
    assert tail % 8 == 0 and (gcs - tail) % 16 == 0 or tail == 0

    def fetch(v_hbm, idx_hbm, off, idx_b, row_b, sem):
        pltpu.async_copy(idx_hbm.at[pl.ds(off, gcs)], idx_b, sem)
        pltpu.async_copy(v_hbm.at[pl.ds(off, gcs)], row_b, sem)

    def await_fetch(v_hbm, idx_hbm, off, idx_b, row_b, sem):
        pltpu.make_async_copy(idx_hbm.at[pl.ds(off, gcs)], idx_b, sem).wait()
        pltpu.make_async_copy(v_hbm.at[pl.ds(off, gcs)], row_b, sem).wait()

    def value_loop(v_hbm, idx_hbm, sid, acc,
                   idx_a, idx_b, row_a, row_b, sem_a, sem_b):
        base = sid * eps
        fetch(v_hbm, idx_hbm, base, idx_a, row_a, sem_a)

        def pair(i, carry):
            off_a = base + 2 * i * gcs
            off_b = off_a + gcs
            fetch(v_hbm, idx_hbm, off_b, idx_b, row_b, sem_b)
            await_fetch(v_hbm, idx_hbm, off_a, idx_a, row_a, sem_a)
            pltpu.sync_copy(row_a, acc.at[idx_a], add=True)

            @pl.when(i < npair - 1)
            def _next_a():
                fetch(v_hbm, idx_hbm, off_b + gcs, idx_a, row_a, sem_a)
            await_fetch(v_hbm, idx_hbm, off_b, idx_b, row_b, sem_b)
            pltpu.sync_copy(row_b, acc.at[idx_b], add=True)
            return carry

        lax.fori_loop(0, npair, pair, 0)
        if nch % 2 == 1:
            off_l = base + (nch - 1) * gcs
            fetch(v_hbm, idx_hbm, off_l, idx_a, row_a, sem_a)
            await_fetch(v_hbm, idx_hbm, off_l, idx_a, row_a, sem_a)
            pltpu.sync_copy(row_a, acc.at[idx_a], add=True)
        if tail:
            off_t = base + nch * gcs
            pltpu.sync_copy(idx_hbm.at[pl.ds(off_t, tail)],
                            idx_a.at[pl.ds(0, tail)])
            for k in range((gcs - tail) // 16):
                idx_a[pl.ds(tail + 16 * k, 16)] = jnp.full(
                    (16,), _PAD_ROW, jnp.int32)
            pltpu.sync_copy(v_hbm.at[pl.ds(off_t, tail)],
                            row_a.at[pl.ds(0, tail)])
            pltpu.sync_copy(row_a, acc.at[idx_a], add=True)

    def count_loop(idx_hbm, sid, acc, idx_a, idx_b, row_a, sem_a, sem_b):
        base = sid * eps
        pltpu.async_copy(idx_hbm.at[pl.ds(base, gcs)], idx_a, sem_a)

        def chunk(c, carry):
            off = base + c * gcs
            noff = off + gcs

            @pl.when((c % 2 == 0) & (c < nch - 1))
            def _pf_b():
                pltpu.async_copy(idx_hbm.at[pl.ds(noff, gcs)], idx_b, sem_b)

            @pl.when(c % 2 == 0)
            def _even():
                pltpu.make_async_copy(
                    idx_hbm.at[pl.ds(off, gcs)], idx_a, sem_a).wait()
                pltpu.sync_copy(row_a, acc.at[idx_a], add=True)

            @pl.when((c % 2 == 1) & (c < nch - 1))
            def _pf_a():
                pltpu.async_copy(idx_hbm.at[pl.ds(noff, gcs)], idx_a, sem_a)

            @pl.when(c % 2 == 1)
            def _odd():
                pltpu.make_async_copy(
                    idx_hbm.at[pl.ds(off, gcs)], idx_b, sem_b).wait()
                pltpu.sync_copy(row_a, acc.at[idx_b], add=True)
            return carry
        lax.fori_loop(0, nch, chunk, 0)
        if tail:
            off_t = base + nch * gcs
            pltpu.sync_copy(idx_hbm.at[pl.ds(off_t, tail)],
                            idx_a.at[pl.ds(0, tail)])
            for k in range((gcs - tail) // 16):
                idx_a[pl.ds(tail + 16 * k, 16)] = jnp.full(
                    (16,), _PAD_ROW, jnp.int32)
            pltpu.sync_copy(row_a, acc.at[idx_a], add=True)

    def body(v1_hbm, v2_hbm, idx_hbm, zz_hbm, out_hbm,
             idx_a, idx_b, row_a, row_b, sem_a, sem_b, acc):
        cid = lax.axis_index("c")
        sid = lax.axis_index("s")
        pltpu.sync_copy(zz_hbm, acc.at[pl.ds(sid * _APS, _APS)])
        plsc.subcore_barrier()

        @pl.when(cid == 0)
        def _core0():
            value_loop(v1_hbm, idx_hbm, sid, acc,
                       idx_a, idx_b, row_a, row_b, sem_a, sem_b)

        @pl.when(cid == 1)
        def _core1():
            if count_mode:
                pltpu.sync_copy(v2_hbm, row_a)
                count_loop(idx_hbm, sid, acc, idx_a, idx_b, row_a,
                           sem_a, sem_b)
            else:
                value_loop(v2_hbm, idx_hbm, sid, acc,
                           idx_a, idx_b, row_a, row_b, sem_a, sem_b)

        plsc.subcore_barrier()
        pltpu.sync_copy(acc.at[pl.ds(sid * _APS, _APS)],
                        out_hbm.at[pl.ds(cid * _ANR + sid * _APS, _APS)])

    def call(v1, v2, idx1):
        zz = jnp.zeros((_APS, 128), jnp.float32)
        k = pl.kernel(
            body,
            out_type=jax.ShapeDtypeStruct((2 * _ANR, 128), jnp.float32),
            mesh=_mesh(),
            scratch_types=[
                pltpu.VMEM((gcs,), jnp.int32),
                pltpu.VMEM((gcs,), jnp.int32),
                pltpu.VMEM((gcs, 128), jnp.float32),
                pltpu.VMEM((gcs, 128), jnp.float32),
                pltpu.SemaphoreType.DMA,
                pltpu.SemaphoreType.DMA,
                pltpu.VMEM_SHARED((_ANR, 128), jnp.float32),
            ],
        )
        return k(v1, v2, idx1, zz)

    return call


_scatter_half = _make_scatter(N_HALF, 176)
_sum_count_half = _make_scatter(N_HALF, 176, count_mode=True)


def _p1_body(n1_ref, n2_ref, ef_ref, wa_ref, wb_ref, wc_ref, b0_ref,
             w1_ref, b1_ref, w2_ref, b2_ref, h3_ref, stats_ref):
    i = pl.program_id(0)
    h = (jnp.dot(n1_ref[...], wa_ref[...], preferred_element_type=jnp.float32)
         + jnp.dot(n2_ref[...], wb_ref[...], preferred_element_type=jnp.float32)
         + jnp.dot(ef_ref[...], wc_ref[...], preferred_element_type=jnp.float32)
         + b0_ref[...])
    h = _leaky(h)
    h = _leaky(jnp.dot(h, w1_ref[...], preferred_element_type=jnp.float32)
               + b1_ref[...])
    h3 = jnp.dot(h, w2_ref[...], preferred_element_type=jnp.float32) + b2_ref[...]
    h3_ref[...] = h3

    @pl.when(i == 0)
    def _init():
        stats_ref[...] = jnp.zeros_like(stats_ref)

    s1 = jnp.sum(h3, axis=0, keepdims=True)
    s2 = jnp.sum(h3 * h3, axis=0, keepdims=True)
    stats_ref[...] += jnp.concatenate(
        [s1, s2, jnp.zeros((6, h3.shape[1]), jnp.float32)], axis=0)


def _edge_pass1(n1, n2, ef, wa, wb, wc, b0, w1, b1, w2, b2):
    nrows = n1.shape[0]
    eb = lambda i: (i, 0)
    fb = lambda i: (0, 0)
    espec = pl.BlockSpec((E_BLK, 128), eb)
    wspec = pl.BlockSpec((128, 128), fb)
    vspec = pl.BlockSpec((1, 128), fb)
    return pl.pallas_call(
        _p1_body,
        grid=(nrows // E_BLK,),
        in_specs=[espec, espec, espec, wspec, wspec, wspec, vspec,
                  wspec, vspec, wspec, vspec],
        out_specs=[espec, pl.BlockSpec((8, 128), fb)],
        out_shape=[
            jax.ShapeDtypeStruct((nrows, 128), jnp.float32),
            jax.ShapeDtypeStruct((8, 128), jnp.float32),
        ],
    )(n1, n2, ef, wa, wb, wc, b0, w1, b1, w2, b2)


def _p2_body(h3_ref, n1_ref, n2_ref, ef_ref, scale_ref, shift_ref,
             wsa_ref, wsb_ref, wsc_ref, bs0_ref, ws1_ref, bs1_ref,
             wma_ref, wmb_ref, wmc_ref, bm0_ref, wm1_ref, bm1_ref,
             enew_ref, v1_ref, v2_ref):
    ek = h3_ref[...] * scale_ref[...] + shift_ref[...]
    enew_ref[...] = ef_ref[...] + ek
    n1 = n1_ref[...]
    n2 = n2_ref[...]
    sh = _leaky(
        jnp.dot(n1, wsa_ref[...], preferred_element_type=jnp.float32)
        + jnp.dot(n2, wsb_ref[...], preferred_element_type=jnp.float32)
        + jnp.dot(ek, wsc_ref[...], preferred_element_type=jnp.float32)
        + bs0_ref[...])
    sij = jnp.dot(sh, ws1_ref[...], preferred_element_type=jnp.float32) + bs1_ref[...]
    w = jnp.exp(jnp.minimum(sij, 60.0))
    mh = _leaky(
        jnp.dot(n1, wma_ref[...], preferred_element_type=jnp.float32)
        + jnp.dot(n2, wmb_ref[...], preferred_element_type=jnp.float32)
        + jnp.dot(ek, wmc_ref[...], preferred_element_type=jnp.float32)
        + bm0_ref[...])
    mij = jnp.dot(mh, wm1_ref[...],
                  preferred_element_type=jnp.float32) + bm1_ref[...]
    v1_ref[...] = w * mij
    v2_ref[...] = w


def _edge_pass2(h3, n1, n2, ef, scale, shift, wsa, wsb, wsc, bs0, ws1, bs1,
                wma, wmb, wmc, bm0, wm1, bm1):
    nrows = n1.shape[0]
    eb = lambda i: (i, 0)
    fb = lambda i: (0, 0)
    espec = pl.BlockSpec((E_BLK, 128), eb)
    wspec = pl.BlockSpec((128, 128), fb)
    vspec = pl.BlockSpec((1, 128), fb)
    return pl.pallas_call(
        _p2_body,
        grid=(nrows // E_BLK,),
        in_specs=[espec, espec, espec, espec, vspec, vspec,
                  wspec, wspec, wspec, vspec, wspec, vspec,
                  wspec, wspec, wspec, vspec, wspec, vspec],
        out_specs=[espec, espec, espec],
        out_shape=[jax.ShapeDtypeStruct((nrows, 128), jnp.float32)] * 3,
    )(h3, n1, n2, ef, scale, shift, wsa, wsb, wsc, bs0, ws1, bs1,
      wma, wmb, wmc, bm0, wm1, bm1)


def _bn_from_stats(stats, n, g, b, eps=1e-5):
    mean = stats[0] / n
    var = stats[1] / n - mean * mean
    scale = g / jnp.sqrt(var + eps)
    shift = b - mean * scale
    return scale[None, :], shift[None, :]


def _attn_layer(layer, nf, ixs, efa, efb):
    i1a, i1b, i2a, i2b = ixs
    w_phi0 = layer["phi_e"][0]["w"]
    w_a0 = layer["fcnna"][0]["w"]
    w_m0 = layer["fcnnm"][0]["w"]

    p1w = (w_phi0[:128], w_phi0[128:256], w_phi0[256:384],
           layer["phi_e"][0]["b"][None, :],
           layer["phi_e"][1]["w"], layer["phi_e"][1]["b"][None, :],
           layer["phi_e"][2]["w"], layer["phi_e"][2]["b"][None, :])
    p2w = (w_a0[:128], w_a0[128:256], w_a0[256:384],
           layer["fcnna"][0]["b"][None, :],
           layer["fcnna"][1]["w"], layer["fcnna"][1]["b"][None, :],
           w_m0[:128], w_m0[128:256], w_m0[256:384],
           layer["fcnnm"][0]["b"][None, :],
           layer["fcnnm"][1]["w"], layer["fcnnm"][1]["b"][None, :])

    n1a, n2a = _gather_half(nf, i1a, i2a)
    n1b, n2b = _gather_half(nf, i1b, i2b)

    h3a, sta = _edge_pass1(n1a, n2a, efa, *p1w)
    h3b, stb = _edge_pass1(n1b, n2b, efb, *p1w)

    scale, shift = _bn_from_stats(sta + stb, float(N_EDGES),
                                  layer["bn1_g"], layer["bn1_b"])

    enewa, v1a, v2a = _edge_pass2(h3a, n1a, n2a, efa, scale, shift, *p2w)
    sega = _scatter_half(v1a, v2a, i1a)
    enewb, v1b, v2b = _edge_pass2(h3b, n1b, n2b, efb, scale, shift, *p2w)
    segb = _scatter_half(v1b, v2b, i1b)

    seg = sega + segb
    msg = seg[:N_NODES] / (seg[_ANR:_ANR + N_NODES] + 1e-16)

    mu = jnp.mean(msg, axis=0)
    var = jnp.var(msg, axis=0)
    nsc = layer["bn2_g"] / jnp.sqrt(var + 1e-5)
    node_new = nf + (msg - mu) * nsc + layer["bn2_b"]
    return node_new, enewa, enewb


def kernel(node_fea, edge_fea, idx1, idx2, idx3, params):
    nf = params["v_emb"][node_fea]
    ef = edge_fea @ params["e_emb"]["w"] + params["e_emb"]["b"]
    efa, efb = ef[:N_HALF], ef[N_HALF:]
    ixs = (idx1[:N_HALF], idx1[N_HALF:], idx2[:N_HALF], idx2[N_HALF:])
    for layer in params["attns"]:
        nf, efa, efb = _attn_layer(layer, nf, ixs, efa, efb)

    ones = jnp.ones((176, 128), jnp.float32)
    pa = _sum_count_half(efa, ones, ixs[0])
    pb = _sum_count_half(efb, ones, ixs[1])
    pooled = pa + pb
    cnt = pooled[_ANR:_ANR + N_NODES, 0]
    vi_e_bar = pooled[:N_NODES] / jnp.maximum(cnt, 1.0)[:, None]
    crys = jnp.concatenate([vi_e_bar, nf], axis=1)
    cnt3 = jax.ops.segment_sum(jnp.ones((N_NODES,), jnp.float32), idx3,
                               num_segments=N_GRAPHS)
    crys = jax.ops.segment_sum(crys, idx3, num_segments=N_GRAPHS)
    crys = crys / jnp.maximum(cnt3, 1.0)[:, None]
    h = _leaky(crys @ params["conv_to_fc"]["w"] + params["conv_to_fc"]["b"])
    for fc in params["fcs"]:
        h = _leaky(h @ fc["w"] + fc["b"])
    return h @ params["fc_out"]["w"] + params["fc_out"]["b"]


# speedup vs baseline: 1.1206x; 1.0187x over previous
import functools

import jax
import jax.numpy as jnp
from jax import lax
from jax.experimental import pallas as pl
from jax.experimental.pallas import tpu as pltpu
from jax.experimental.pallas import tpu_sc as plsc

N_NODES = 10000
N_EDGES = 320000
N_HALF = N_EDGES // 2
N_GRAPHS = 256
E_BLK = 3200

_NC = 2
_NS = 16
_NW = _NC * _NS
_ANR = 10240
_APS = _ANR // _NS


def _leaky(x):
    return jnp.where(x >= 0, x, 0.2 * x)


def _mesh():
    return plsc.VectorSubcoreMesh(core_axis_name="c", subcore_axis_name="s")


def _make_gather(n_rows, gc, nb=4):
    epw = n_rows // _NW
    nch = epw // gc
    ngrp = (nch - 1) // nb

    def body(nf_hbm, idx1_hbm, idx2_hbm, n1_hbm, n2_hbm,
             idx1_v, idx2_v, b0, b1, b2, b3,
             g0, g1, g2, g3, w0, w1, w2, w3):
        bufs = (b0, b1, b2, b3)
        gs = (g0, g1, g2, g3)
        ws = (w0, w1, w2, w3)
        cid = lax.axis_index("c")
        sid = lax.axis_index("s")
        wid = sid * _NC + cid
        base = wid * epw
        pltpu.sync_copy(idx1_hbm.at[pl.ds(base, epw)], idx1_v)
        pltpu.sync_copy(idx2_hbm.at[pl.ds(base, epw)], idx2_v)

        def phase(idx_v, out_hbm):
            def start_g(b, c):
                pltpu.async_copy(
                    nf_hbm.at[idx_v.at[pl.ds(c * gc, gc)]], bufs[b], gs[b])

            def wait_g(b, c):
                pltpu.make_async_copy(
                    nf_hbm.at[idx_v.at[pl.ds(c * gc, gc)]],
                    bufs[b], gs[b]).wait()

            def start_w(b, c):
                pltpu.async_copy(
                    bufs[b], out_hbm.at[pl.ds(base + c * gc, gc)], ws[b])

            def wait_w(b, c):
                pltpu.make_async_copy(
                    bufs[b], out_hbm.at[pl.ds(base + c * gc, gc)],
                    ws[b]).wait()

            start_g(0, 0)
            start_g(1, 1)

            def group(i, carry):
                c0 = i * nb
                for j in range(nb):
                    c = c0 + j
                    bn = (j + 2) % nb

                    @pl.when(c + 2 < nch)
                    def _pf():
                        @pl.when(c >= 2)
                        def _dr():
                            wait_w(bn, c - 2)
                        start_g(bn, c + 2)
                    wait_g(j, c)
                    start_w(j, c)
                return carry

            lax.fori_loop(0, ngrp, group, 0)
            for c in range(ngrp * nb, nch):
                wait_g(c % nb, c)
                start_w(c % nb, c)
            for c in range(nch - nb, nch):
                wait_w(c % nb, c)

        phase(idx1_v, n1_hbm)
        phase(idx2_v, n2_hbm)

    def call(nf, idx1, idx2):
        k = pl.kernel(
            body,
            out_type=[jax.ShapeDtypeStruct((n_rows, 128), jnp.float32)] * 2,
            mesh=_mesh(),
            scratch_types=[
                pltpu.VMEM((epw,), jnp.int32),
                pltpu.VMEM((epw,), jnp.int32),
            ] + [pltpu.VMEM((gc, 128), jnp.float32)] * 4
            + [pltpu.SemaphoreType.DMA] * 8,
        )
        return k(nf, idx1, idx2)

    return call


_gather_half = _make_gather(N_HALF, 200)


_PAD_ROW = 10200


def _make_scatter(n_rows, gcs, count_mode=False):
    eps = n_rows // _NS
    nch = eps // gcs
    npair = nch // 2
    tail = eps - nch * gcs
    assert tail % 8 == 0 and (gcs - tail) % 16 == 0 or tail == 0

    def fetch(v_hbm, idx_hbm, off, idx_b, row_b, sem):
        pltpu.async_copy(idx_hbm.at[pl.ds(off, gcs)], idx_b, sem)
        pltpu.async_copy(v_hbm.at[pl.ds(off, gcs)], row_b, sem)

    def await_fetch(v_hbm, idx_hbm, off, idx_b, row_b, sem):
        pltpu.make_async_copy(idx_hbm.at[pl.ds(off, gcs)], idx_b, sem).wait()
        pltpu.make_async_copy(v_hbm.at[pl.ds(off, gcs)], row_b, sem).wait()

    def value_loop(v_hbm, idx_hbm, sid, acc,
                   idx_a, idx_b, row_a, row_b, sem_a, sem_b):
        base = sid * eps
        fetch(v_hbm, idx_hbm, base, idx_a, row_a, sem_a)

        def pair(i, carry):
            off_a = base + 2 * i * gcs
            off_b = off_a + gcs
            fetch(v_hbm, idx_hbm, off_b, idx_b, row_b, sem_b)
            await_fetch(v_hbm, idx_hbm, off_a, idx_a, row_a, sem_a)
            pltpu.sync_copy(row_a, acc.at[idx_a], add=True)

            @pl.when(i < npair - 1)
            def _next_a():
                fetch(v_hbm, idx_hbm, off_b + gcs, idx_a, row_a, sem_a)
            await_fetch(v_hbm, idx_hbm, off_b, idx_b, row_b, sem_b)
            pltpu.sync_copy(row_b, acc.at[idx_b], add=True)
            return carry

        lax.fori_loop(0, npair, pair, 0)
        if nch % 2 == 1:
            off_l = base + (nch - 1) * gcs
            fetch(v_hbm, idx_hbm, off_l, idx_a, row_a, sem_a)
            await_fetch(v_hbm, idx_hbm, off_l, idx_a, row_a, sem_a)
            pltpu.sync_copy(row_a, acc.at[idx_a], add=True)
        if tail:
            off_t = base + nch * gcs
            pltpu.sync_copy(idx_hbm.at[pl.ds(off_t, tail)],
                            idx_a.at[pl.ds(0, tail)])
            for k in range((gcs - tail) // 16):
                idx_a[pl.ds(tail + 16 * k, 16)] = jnp.full(
                    (16,), _PAD_ROW, jnp.int32)
            pltpu.sync_copy(v_hbm.at[pl.ds(off_t, tail)],
                            row_a.at[pl.ds(0, tail)])
            pltpu.sync_copy(row_a, acc.at[idx_a], add=True)

    def count_loop(idx_hbm, sid, acc, idx_a, idx_b, row_a, sem_a, sem_b):
        base = sid * eps
        pltpu.async_copy(idx_hbm.at[pl.ds(base, gcs)], idx_a, sem_a)

        def chunk(c, carry):
            off = base + c * gcs
            noff = off + gcs

            @pl.when((c % 2 == 0) & (c < nch - 1))
            def _pf_b():
                pltpu.async_copy(idx_hbm.at[pl.ds(noff, gcs)], idx_b, sem_b)

            @pl.when(c % 2 == 0)
            def _even():
                pltpu.make_async_copy(
                    idx_hbm.at[pl.ds(off, gcs)], idx_a, sem_a).wait()
                pltpu.sync_copy(row_a, acc.at[idx_a], add=True)

            @pl.when((c % 2 == 1) & (c < nch - 1))
            def _pf_a():
                pltpu.async_copy(idx_hbm.at[pl.ds(noff, gcs)], idx_a, sem_a)

            @pl.when(c % 2 == 1)
            def _odd():
                pltpu.make_async_copy(
                    idx_hbm.at[pl.ds(off, gcs)], idx_b, sem_b).wait()
                pltpu.sync_copy(row_a, acc.at[idx_b], add=True)
            return carry
        lax.fori_loop(0, nch, chunk, 0)
        if tail:
            off_t = base + nch * gcs
            pltpu.sync_copy(idx_hbm.at[pl.ds(off_t, tail)],
                            idx_a.at[pl.ds(0, tail)])
            for k in range((gcs - tail) // 16):
                idx_a[pl.ds(tail + 16 * k, 16)] = jnp.full(
                    (16,), _PAD_ROW, jnp.int32)
            pltpu.sync_copy(row_a, acc.at[idx_a], add=True)

    def body(v1_hbm, v2_hbm, idx_hbm, zz_hbm, out_hbm,
             idx_a, idx_b, row_a, row_b, sem_a, sem_b, acc):
        cid = lax.axis_index("c")
        sid = lax.axis_index("s")
        pltpu.sync_copy(zz_hbm, acc.at[pl.ds(sid * _APS, _APS)])
        plsc.subcore_barrier()

        @pl.when(cid == 0)
        def _core0():
            value_loop(v1_hbm, idx_hbm, sid, acc,
                       idx_a, idx_b, row_a, row_b, sem_a, sem_b)

        @pl.when(cid == 1)
        def _core1():
            if count_mode:
                pltpu.sync_copy(v2_hbm, row_a)
                count_loop(idx_hbm, sid, acc, idx_a, idx_b, row_a,
                           sem_a, sem_b)
            else:
                value_loop(v2_hbm, idx_hbm, sid, acc,
                           idx_a, idx_b, row_a, row_b, sem_a, sem_b)

        plsc.subcore_barrier()
        pltpu.sync_copy(acc.at[pl.ds(sid * _APS, _APS)],
                        out_hbm.at[pl.ds(cid * _ANR + sid * _APS, _APS)])

    def call(v1, v2, idx1):
        zz = jnp.zeros((_APS, 128), jnp.float32)
        k = pl.kernel(
            body,
            out_type=jax.ShapeDtypeStruct((2 * _ANR, 128), jnp.float32),
            mesh=_mesh(),
            scratch_types=[
                pltpu.VMEM((gcs,), jnp.int32),
                pltpu.VMEM((gcs,), jnp.int32),
                pltpu.VMEM((gcs, 128), jnp.float32),
                pltpu.VMEM((gcs, 128), jnp.float32),
                pltpu.SemaphoreType.DMA,
                pltpu.SemaphoreType.DMA,
                pltpu.VMEM_SHARED((_ANR, 128), jnp.float32),
            ],
        )
        return k(v1, v2, idx1, zz)

    return call


_scatter_half = _make_scatter(N_HALF, 176)
_sum_count_half = _make_scatter(N_HALF, 176, count_mode=True)


def _p1_body(n1_ref, n2_ref, ef_ref, wa_ref, wb_ref, wc_ref, b0_ref,
             w1_ref, b1_ref, w2_ref, b2_ref, h3_ref, stats_ref):
    i = pl.program_id(0)
    h = (jnp.dot(n1_ref[...], wa_ref[...], preferred_element_type=jnp.float32)
         + jnp.dot(n2_ref[...], wb_ref[...], preferred_element_type=jnp.float32)
         + jnp.dot(ef_ref[...], wc_ref[...], preferred_element_type=jnp.float32)
         + b0_ref[...])
    h = _leaky(h)
    h = _leaky(jnp.dot(h, w1_ref[...], preferred_element_type=jnp.float32)
               + b1_ref[...])
    h3 = jnp.dot(h, w2_ref[...], preferred_element_type=jnp.float32) + b2_ref[...]
    h3_ref[...] = h3

    @pl.when(i == 0)
    def _init():
        stats_ref[...] = jnp.zeros_like(stats_ref)

    s1 = jnp.sum(h3, axis=0, keepdims=True)
    s2 = jnp.sum(h3 * h3, axis=0, keepdims=True)
    stats_ref[...] += jnp.concatenate(
        [s1, s2, jnp.zeros((6, h3.shape[1]), jnp.float32)], axis=0)


def _edge_pass1(n1, n2, ef, wa, wb, wc, b0, w1, b1, w2, b2):
    nrows = n1.shape[0]
    eb = lambda i: (i, 0)
    fb = lambda i: (0, 0)
    espec = pl.BlockSpec((E_BLK, 128), eb)
    wspec = pl.BlockSpec((128, 128), fb)
    vspec = pl.BlockSpec((1, 128), fb)
    return pl.pallas_call(
        _p1_body,
        grid=(nrows // E_BLK,),
        in_specs=[espec, espec, espec, wspec, wspec, wspec, vspec,
                  wspec, vspec, wspec, vspec],
        out_specs=[espec, pl.BlockSpec((8, 128), fb)],
        out_shape=[
            jax.ShapeDtypeStruct((nrows, 128), jnp.float32),
            jax.ShapeDtypeStruct((8, 128), jnp.float32),
        ],
    )(n1, n2, ef, wa, wb, wc, b0, w1, b1, w2, b2)


def _p2_body(h3_ref, n1_ref, n2_ref, ef_ref, scale_ref, shift_ref,
             wsa_ref, wsb_ref, wsc_ref, bs0_ref, ws1_ref, bs1_ref,
             wma_ref, wmb_ref, wmc_ref, bm0_ref, wm1_ref, bm1_ref,
             enew_ref, v1_ref, v2_ref):
    ek = h3_ref[...] * scale_ref[...] + shift_ref[...]
    enew_ref[...] = ef_ref[...] + ek
    n1 = n1_ref[...]
    n2 = n2_ref[...]
    sh = _leaky(
        jnp.dot(n1, wsa_ref[...], preferred_element_type=jnp.float32)
        + jnp.dot(n2, wsb_ref[...], preferred_element_type=jnp.float32)
        + jnp.dot(ek, wsc_ref[...], preferred_element_type=jnp.float32)
        + bs0_ref[...])
    sij = jnp.dot(sh, ws1_ref[...], preferred_element_type=jnp.float32) + bs1_ref[...]
    w = jnp.exp(jnp.minimum(sij, 60.0))
    mh = _leaky(
        jnp.dot(n1, wma_ref[...], preferred_element_type=jnp.float32)
        + jnp.dot(n2, wmb_ref[...], preferred_element_type=jnp.float32)
        + jnp.dot(ek, wmc_ref[...], preferred_element_type=jnp.float32)
        + bm0_ref[...])
    mij = jnp.dot(mh, wm1_ref[...],
                  preferred_element_type=jnp.float32) + bm1_ref[...]
    v1_ref[...] = w * mij
    v2_ref[...] = w


def _edge_pass2(h3, n1, n2, ef, scale, shift, wsa, wsb, wsc, bs0, ws1, bs1,
                wma, wmb, wmc, bm0, wm1, bm1):
    nrows = n1.shape[0]
    eb = lambda i: (i, 0)
    fb = lambda i: (0, 0)
    espec = pl.BlockSpec((E_BLK, 128), eb)
    wspec = pl.BlockSpec((128, 128), fb)
    vspec = pl.BlockSpec((1, 128), fb)
    return pl.pallas_call(
        _p2_body,
        grid=(nrows // E_BLK,),
        in_specs=[espec, espec, espec, espec, vspec, vspec,
                  wspec, wspec, wspec, vspec, wspec, vspec,
                  wspec, wspec, wspec, vspec, wspec, vspec],
        out_specs=[espec, espec, espec],
        out_shape=[jax.ShapeDtypeStruct((nrows, 128), jnp.float32)] * 3,
    )(h3, n1, n2, ef, scale, shift, wsa, wsb, wsc, bs0, ws1, bs1,
      wma, wmb, wmc, bm0, wm1, bm1)


def _bn_from_stats(stats, n, g, b, eps=1e-5):
    mean = stats[0] / n
    var = stats[1] / n - mean * mean
    scale = g / jnp.sqrt(var + eps)
    shift = b - mean * scale
    return scale[None, :], shift[None, :]


def _attn_layer(layer, nf, ixs, efa, efb):
    i1a, i1b, i2a, i2b = ixs
    w_phi0 = layer["phi_e"][0]["w"]
    w_a0 = layer["fcnna"][0]["w"]
    w_m0 = layer["fcnnm"][0]["w"]

    p1w = (w_phi0[:128], w_phi0[128:256], w_phi0[256:384],
           layer["phi_e"][0]["b"][None, :],
           layer["phi_e"][1]["w"], layer["phi_e"][1]["b"][None, :],
           layer["phi_e"][2]["w"], layer["phi_e"][2]["b"][None, :])
    p2w = (w_a0[:128], w_a0[128:256], w_a0[256:384],
           layer["fcnna"][0]["b"][None, :],
           layer["fcnna"][1]["w"], layer["fcnna"][1]["b"][None, :],
           w_m0[:128], w_m0[128:256], w_m0[256:384],
           layer["fcnnm"][0]["b"][None, :],
           layer["fcnnm"][1]["w"], layer["fcnnm"][1]["b"][None, :])

    n1a, n2a = _gather_half(nf, i1a, i2a)
    n1b, n2b = _gather_half(nf, i1b, i2b)

    h3a, sta = _edge_pass1(n1a, n2a, efa, *p1w)
    h3b, stb = _edge_pass1(n1b, n2b, efb, *p1w)

    scale, shift = _bn_from_stats(sta + stb, float(N_EDGES),
                                  layer["bn1_g"], layer["bn1_b"])

    enewa, v1a, v2a = _edge_pass2(h3a, n1a, n2a, efa, scale, shift, *p2w)
    sega = _scatter_half(v1a, v2a, i1a)
    enewb, v1b, v2b = _edge_pass2(h3b, n1b, n2b, efb, scale, shift, *p2w)
    segb = _scatter_half(v1b, v2b, i1b)

    seg = sega + segb
    msg = seg[:N_NODES] / (seg[_ANR:_ANR + N_NODES] + 1e-16)

    mu = jnp.mean(msg, axis=0)
    var = jnp.var(msg, axis=0)
    nsc = layer["bn2_g"] / jnp.sqrt(var + 1e-5)
    node_new = nf + (msg - mu) * nsc + layer["bn2_b"]
    return node_new, enewa, enewb


def kernel(node_fea, edge_fea, idx1, idx2, idx3, params):
    nf = params["v_emb"][node_fea]
    we, be = params["e_emb"]["w"], params["e_emb"]["b"]
    efa = edge_fea[:N_HALF] @ we + be
    efb = edge_fea[N_HALF:] @ we + be
    ixs = (idx1[:N_HALF], idx1[N_HALF:], idx2[:N_HALF], idx2[N_HALF:])
    for layer in params["attns"]:
        nf, efa, efb = _attn_layer(layer, nf, ixs, efa, efb)

    ones = jnp.ones((176, 128), jnp.float32)
    pa = _sum_count_half(efa, ones, ixs[0])
    pb = _sum_count_half(efb, ones, ixs[1])
    pooled = pa + pb
    cnt = pooled[_ANR:_ANR + N_NODES, 0]
    vi_e_bar = pooled[:N_NODES] / jnp.maximum(cnt, 1.0)[:, None]
    crys = jnp.concatenate([vi_e_bar, nf], axis=1)
    cnt3 = jax.ops.segment_sum(jnp.ones((N_NODES,), jnp.float32), idx3,
                               num_segments=N_GRAPHS)
    crys = jax.ops.segment_sum(crys, idx3, num_segments=N_GRAPHS)
    crys = crys / jnp.maximum(cnt3, 1.0)[:, None]
    h = _leaky(crys @ params["conv_to_fc"]["w"] + params["conv_to_fc"]["b"])
    for fc in params["fcs"]:
        h = _leaky(h @ fc["w"] + fc["b"])
    return h @ params["fc_out"]["w"] + params["fc_out"]["b"]


# speedup vs baseline: 1.1215x; 1.0009x over previous
import functools

import jax
import jax.numpy as jnp
from jax import lax
from jax.experimental import pallas as pl
from jax.experimental.pallas import tpu as pltpu
from jax.experimental.pallas import tpu_sc as plsc

N_NODES = 10000
N_EDGES = 320000
N_HALF = N_EDGES // 2
N_GRAPHS = 256
E_BLK = 3200

_NC = 2
_NS = 16
_NW = _NC * _NS
_ANR = 10240
_APS = _ANR // _NS


def _leaky(x):
    return jnp.where(x >= 0, x, 0.2 * x)


def _mesh():
    return plsc.VectorSubcoreMesh(core_axis_name="c", subcore_axis_name="s")


def _make_gather(n_rows, gc, nb=4):
    epw = n_rows // _NW
    nch = epw // gc
    ngrp = (nch - 1) // nb

    def body(nf_hbm, idx1_hbm, idx2_hbm, n1_hbm, n2_hbm,
             idx1_v, idx2_v, b0, b1, b2, b3,
             g0, g1, g2, g3, w0, w1, w2, w3):
        bufs = (b0, b1, b2, b3)
        gs = (g0, g1, g2, g3)
        ws = (w0, w1, w2, w3)
        cid = lax.axis_index("c")
        sid = lax.axis_index("s")
        wid = sid * _NC + cid
        base = wid * epw
        pltpu.sync_copy(idx1_hbm.at[pl.ds(base, epw)], idx1_v)
        pltpu.sync_copy(idx2_hbm.at[pl.ds(base, epw)], idx2_v)

        def phase(idx_v, out_hbm):
            def start_g(b, c):
                pltpu.async_copy(
                    nf_hbm.at[idx_v.at[pl.ds(c * gc, gc)]], bufs[b], gs[b])

            def wait_g(b, c):
                pltpu.make_async_copy(
                    nf_hbm.at[idx_v.at[pl.ds(c * gc, gc)]],
                    bufs[b], gs[b]).wait()

            def start_w(b, c):
                pltpu.async_copy(
                    bufs[b], out_hbm.at[pl.ds(base + c * gc, gc)], ws[b])

            def wait_w(b, c):
                pltpu.make_async_copy(
                    bufs[b], out_hbm.at[pl.ds(base + c * gc, gc)],
                    ws[b]).wait()

            start_g(0, 0)
            start_g(1, 1)

            def group(i, carry):
                c0 = i * nb
                for j in range(nb):
                    c = c0 + j
                    bn = (j + 2) % nb

                    @pl.when(c + 2 < nch)
                    def _pf():
                        @pl.when(c >= 2)
                        def _dr():
                            wait_w(bn, c - 2)
                        start_g(bn, c + 2)
                    wait_g(j, c)
                    start_w(j, c)
                return carry

            lax.fori_loop(0, ngrp, group, 0)
            for c in range(ngrp * nb, nch):
                wait_g(c % nb, c)
                start_w(c % nb, c)
            for c in range(nch - nb, nch):
                wait_w(c % nb, c)

        phase(idx1_v, n1_hbm)
        phase(idx2_v, n2_hbm)

    def call(nf, idx1, idx2):
        k = pl.kernel(
            body,
            out_type=[jax.ShapeDtypeStruct((n_rows, 128), jnp.float32)] * 2,
            mesh=_mesh(),
            scratch_types=[
                pltpu.VMEM((epw,), jnp.int32),
                pltpu.VMEM((epw,), jnp.int32),
            ] + [pltpu.VMEM((gc, 128), jnp.float32)] * 4
            + [pltpu.SemaphoreType.DMA] * 8,
        )
        return k(nf, idx1, idx2)

    return call


_gather_half = _make_gather(N_HALF, 200)


_PAD_ROW = 10200


def _make_scatter(n_rows, gcs, count_mode=False):
    eps = n_rows // _NS
    nch = eps // gcs
    npair = nch // 2
    tail = eps - nch * gcs
    assert tail % 8 == 0 and (gcs - tail) % 16 == 0 or tail == 0

    def fetch(v_hbm, idx_hbm, off, idx_b, row_b, sem):
        pltpu.async_copy(idx_hbm.at[pl.ds(off, gcs)], idx_b, sem)
        pltpu.async_copy(v_hbm.at[pl.ds(off, gcs)], row_b, sem)

    def await_fetch(v_hbm, idx_hbm, off, idx_b, row_b, sem):
        pltpu.make_async_copy(idx_hbm.at[pl.ds(off, gcs)], idx_b, sem).wait()
        pltpu.make_async_copy(v_hbm.at[pl.ds(off, gcs)], row_b, sem).wait()

    def value_loop(v_hbm, idx_hbm, sid, acc,
                   idx_a, idx_b, row_a, row_b, sem_a, sem_b):
        base = sid * eps
        fetch(v_hbm, idx_hbm, base, idx_a, row_a, sem_a)

        def pair(i, carry):
            off_a = base + 2 * i * gcs
            off_b = off_a + gcs
            fetch(v_hbm, idx_hbm, off_b, idx_b, row_b, sem_b)
            await_fetch(v_hbm, idx_hbm, off_a, idx_a, row_a, sem_a)
            pltpu.sync_copy(row_a, acc.at[idx_a], add=True)

            @pl.when(i < npair - 1)
            def _next_a():
                fetch(v_hbm, idx_hbm, off_b + gcs, idx_a, row_a, sem_a)
            await_fetch(v_hbm, idx_hbm, off_b, idx_b, row_b, sem_b)
            pltpu.sync_copy(row_b, acc.at[idx_b], add=True)
            return carry

        lax.fori_loop(0, npair, pair, 0)
        if nch % 2 == 1:
            off_l = base + (nch - 1) * gcs
            fetch(v_hbm, idx_hbm, off_l, idx_a, row_a, sem_a)
            await_fetch(v_hbm, idx_hbm, off_l, idx_a, row_a, sem_a)
            pltpu.sync_copy(row_a, acc.at[idx_a], add=True)
        if tail:
            off_t = base + nch * gcs
            pltpu.sync_copy(idx_hbm.at[pl.ds(off_t, tail)],
                            idx_a.at[pl.ds(0, tail)])
            for k in range((gcs - tail) // 16):
                idx_a[pl.ds(tail + 16 * k, 16)] = jnp.full(
                    (16,), _PAD_ROW, jnp.int32)
            pltpu.sync_copy(v_hbm.at[pl.ds(off_t, tail)],
                            row_a.at[pl.ds(0, tail)])
            pltpu.sync_copy(row_a, acc.at[idx_a], add=True)

    def count_loop(idx_hbm, sid, acc, idx_a, idx_b, row_a, sem_a, sem_b):
        base = sid * eps
        pltpu.async_copy(idx_hbm.at[pl.ds(base, gcs)], idx_a, sem_a)

        def chunk(c, carry):
            off = base + c * gcs
            noff = off + gcs

            @pl.when((c % 2 == 0) & (c < nch - 1))
            def _pf_b():
                pltpu.async_copy(idx_hbm.at[pl.ds(noff, gcs)], idx_b, sem_b)

            @pl.when(c % 2 == 0)
            def _even():
                pltpu.make_async_copy(
                    idx_hbm.at[pl.ds(off, gcs)], idx_a, sem_a).wait()
                pltpu.sync_copy(row_a, acc.at[idx_a], add=True)

            @pl.when((c % 2 == 1) & (c < nch - 1))
            def _pf_a():
                pltpu.async_copy(idx_hbm.at[pl.ds(noff, gcs)], idx_a, sem_a)

            @pl.when(c % 2 == 1)
            def _odd():
                pltpu.make_async_copy(
                    idx_hbm.at[pl.ds(off, gcs)], idx_b, sem_b).wait()
                pltpu.sync_copy(row_a, acc.at[idx_b], add=True)
            return carry
        lax.fori_loop(0, nch, chunk, 0)
        if tail:
            off_t = base + nch * gcs
            pltpu.sync_copy(idx_hbm.at[pl.ds(off_t, tail)],
                            idx_a.at[pl.ds(0, tail)])
            for k in range((gcs - tail) // 16):
                idx_a[pl.ds(tail + 16 * k, 16)] = jnp.full(
                    (16,), _PAD_ROW, jnp.int32)
            pltpu.sync_copy(row_a, acc.at[idx_a], add=True)

    def body(v1_hbm, v2_hbm, idx_hbm, zz_hbm, out_hbm,
             idx_a, idx_b, row_a, row_b, sem_a, sem_b, acc):
        cid = lax.axis_index("c")
        sid = lax.axis_index("s")
        pltpu.sync_copy(zz_hbm, acc.at[pl.ds(sid * _APS, _APS)])
        plsc.subcore_barrier()

        @pl.when(cid == 0)
        def _core0():
            value_loop(v1_hbm, idx_hbm, sid, acc,
                       idx_a, idx_b, row_a, row_b, sem_a, sem_b)

        @pl.when(cid == 1)
        def _core1():
            if count_mode:
                pltpu.sync_copy(v2_hbm, row_a)
                count_loop(idx_hbm, sid, acc, idx_a, idx_b, row_a,
                           sem_a, sem_b)
            else:
                value_loop(v2_hbm, idx_hbm, sid, acc,
                           idx_a, idx_b, row_a, row_b, sem_a, sem_b)

        plsc.subcore_barrier()
        pltpu.sync_copy(acc.at[pl.ds(sid * _APS, _APS)],
                        out_hbm.at[pl.ds(cid * _ANR + sid * _APS, _APS)])

    def call(v1, v2, idx1):
        zz = jnp.zeros((_APS, 128), jnp.float32)
        k = pl.kernel(
            body,
            out_type=jax.ShapeDtypeStruct((2 * _ANR, 128), jnp.float32),
            mesh=_mesh(),
            scratch_types=[
                pltpu.VMEM((gcs,), jnp.int32),
                pltpu.VMEM((gcs,), jnp.int32),
                pltpu.VMEM((gcs, 128), jnp.float32),
                pltpu.VMEM((gcs, 128), jnp.float32),
                pltpu.SemaphoreType.DMA,
                pltpu.SemaphoreType.DMA,
                pltpu.VMEM_SHARED((_ANR, 128), jnp.float32),
            ],
        )
        return k(v1, v2, idx1, zz)

    return call


_scatter_half = _make_scatter(N_HALF, 176)
_sum_count_half = _make_scatter(N_HALF, 176, count_mode=True)


def _p1_body(n1_ref, n2_ref, ef_ref, wa_ref, wb_ref, wc_ref, b0_ref,
             w1_ref, b1_ref, w2_ref, b2_ref, h3_ref, stats_ref):
    i = pl.program_id(0)
    h = (jnp.dot(n1_ref[...], wa_ref[...], preferred_element_type=jnp.float32, precision=lax.Precision.DEFAULT)
         + jnp.dot(n2_ref[...], wb_ref[...], preferred_element_type=jnp.float32, precision=lax.Precision.DEFAULT)
         + jnp.dot(ef_ref[...], wc_ref[...], preferred_element_type=jnp.float32, precision=lax.Precision.DEFAULT)
         + b0_ref[...])
    h = _leaky(h)
    h = _leaky(jnp.dot(h, w1_ref[...], preferred_element_type=jnp.float32, precision=lax.Precision.DEFAULT)
               + b1_ref[...])
    h3 = jnp.dot(h, w2_ref[...], preferred_element_type=jnp.float32, precision=lax.Precision.DEFAULT) + b2_ref[...]
    h3_ref[...] = h3

    @pl.when(i == 0)
    def _init():
        stats_ref[...] = jnp.zeros_like(stats_ref)

    s1 = jnp.sum(h3, axis=0, keepdims=True)
    s2 = jnp.sum(h3 * h3, axis=0, keepdims=True)
    stats_ref[...] += jnp.concatenate(
        [s1, s2, jnp.zeros((6, h3.shape[1]), jnp.float32)], axis=0)


def _edge_pass1(n1, n2, ef, wa, wb, wc, b0, w1, b1, w2, b2):
    nrows = n1.shape[0]
    eb = lambda i: (i, 0)
    fb = lambda i: (0, 0)
    espec = pl.BlockSpec((E_BLK, 128), eb)
    wspec = pl.BlockSpec((128, 128), fb)
    vspec = pl.BlockSpec((1, 128), fb)
    return pl.pallas_call(
        _p1_body,
        grid=(nrows // E_BLK,),
        in_specs=[espec, espec, espec, wspec, wspec, wspec, vspec,
                  wspec, vspec, wspec, vspec],
        out_specs=[espec, pl.BlockSpec((8, 128), fb)],
        out_shape=[
            jax.ShapeDtypeStruct((nrows, 128), jnp.float32),
            jax.ShapeDtypeStruct((8, 128), jnp.float32),
        ],
    )(n1, n2, ef, wa, wb, wc, b0, w1, b1, w2, b2)


def _p2_body(h3_ref, n1_ref, n2_ref, ef_ref, scale_ref, shift_ref,
             wsa_ref, wsb_ref, wsc_ref, bs0_ref, ws1_ref, bs1_ref,
             wma_ref, wmb_ref, wmc_ref, bm0_ref, wm1_ref, bm1_ref,
             enew_ref, v1_ref, v2_ref):
    ek = h3_ref[...] * scale_ref[...] + shift_ref[...]
    enew_ref[...] = ef_ref[...] + ek
    n1 = n1_ref[...]
    n2 = n2_ref[...]
    sh = _leaky(
        jnp.dot(n1, wsa_ref[...], preferred_element_type=jnp.float32, precision=lax.Precision.DEFAULT)
        + jnp.dot(n2, wsb_ref[...], preferred_element_type=jnp.float32, precision=lax.Precision.DEFAULT)
        + jnp.dot(ek, wsc_ref[...], preferred_element_type=jnp.float32, precision=lax.Precision.DEFAULT)
        + bs0_ref[...])
    sij = jnp.dot(sh, ws1_ref[...], preferred_element_type=jnp.float32, precision=lax.Precision.DEFAULT) + bs1_ref[...]
    w = jnp.exp(jnp.minimum(sij, 60.0))
    mh = _leaky(
        jnp.dot(n1, wma_ref[...], preferred_element_type=jnp.float32, precision=lax.Precision.DEFAULT)
        + jnp.dot(n2, wmb_ref[...], preferred_element_type=jnp.float32, precision=lax.Precision.DEFAULT)
        + jnp.dot(ek, wmc_ref[...], preferred_element_type=jnp.float32, precision=lax.Precision.DEFAULT)
        + bm0_ref[...])
    mij = jnp.dot(mh, wm1_ref[...],
                  preferred_element_type=jnp.float32, precision=lax.Precision.DEFAULT) + bm1_ref[...]
    v1_ref[...] = w * mij
    v2_ref[...] = w


def _edge_pass2(h3, n1, n2, ef, scale, shift, wsa, wsb, wsc, bs0, ws1, bs1,
                wma, wmb, wmc, bm0, wm1, bm1):
    nrows = n1.shape[0]
    eb = lambda i: (i, 0)
    fb = lambda i: (0, 0)
    espec = pl.BlockSpec((E_BLK, 128), eb)
    wspec = pl.BlockSpec((128, 128), fb)
    vspec = pl.BlockSpec((1, 128), fb)
    return pl.pallas_call(
        _p2_body,
        grid=(nrows // E_BLK,),
        in_specs=[espec, espec, espec, espec, vspec, vspec,
                  wspec, wspec, wspec, vspec, wspec, vspec,
                  wspec, wspec, wspec, vspec, wspec, vspec],
        out_specs=[espec, espec, espec],
        out_shape=[jax.ShapeDtypeStruct((nrows, 128), jnp.float32)] * 3,
    )(h3, n1, n2, ef, scale, shift, wsa, wsb, wsc, bs0, ws1, bs1,
      wma, wmb, wmc, bm0, wm1, bm1)


def _bn_from_stats(stats, n, g, b, eps=1e-5):
    mean = stats[0] / n
    var = stats[1] / n - mean * mean
    scale = g / jnp.sqrt(var + eps)
    shift = b - mean * scale
    return scale[None, :], shift[None, :]


def _attn_layer(layer, nf, ixs, efa, efb):
    i1a, i1b, i2a, i2b = ixs
    w_phi0 = layer["phi_e"][0]["w"]
    w_a0 = layer["fcnna"][0]["w"]
    w_m0 = layer["fcnnm"][0]["w"]

    p1w = (w_phi0[:128], w_phi0[128:256], w_phi0[256:384],
           layer["phi_e"][0]["b"][None, :],
           layer["phi_e"][1]["w"], layer["phi_e"][1]["b"][None, :],
           layer["phi_e"][2]["w"], layer["phi_e"][2]["b"][None, :])
    p2w = (w_a0[:128], w_a0[128:256], w_a0[256:384],
           layer["fcnna"][0]["b"][None, :],
           layer["fcnna"][1]["w"], layer["fcnna"][1]["b"][None, :],
           w_m0[:128], w_m0[128:256], w_m0[256:384],
           layer["fcnnm"][0]["b"][None, :],
           layer["fcnnm"][1]["w"], layer["fcnnm"][1]["b"][None, :])

    n1a, n2a = _gather_half(nf, i1a, i2a)
    n1b, n2b = _gather_half(nf, i1b, i2b)

    h3a, sta = _edge_pass1(n1a, n2a, efa, *p1w)
    h3b, stb = _edge_pass1(n1b, n2b, efb, *p1w)

    scale, shift = _bn_from_stats(sta + stb, float(N_EDGES),
                                  layer["bn1_g"], layer["bn1_b"])

    enewa, v1a, v2a = _edge_pass2(h3a, n1a, n2a, efa, scale, shift, *p2w)
    sega = _scatter_half(v1a, v2a, i1a)
    enewb, v1b, v2b = _edge_pass2(h3b, n1b, n2b, efb, scale, shift, *p2w)
    segb = _scatter_half(v1b, v2b, i1b)

    seg = sega + segb
    msg = seg[:N_NODES] / (seg[_ANR:_ANR + N_NODES] + 1e-16)

    mu = jnp.mean(msg, axis=0)
    var = jnp.var(msg, axis=0)
    nsc = layer["bn2_g"] / jnp.sqrt(var + 1e-5)
    node_new = nf + (msg - mu) * nsc + layer["bn2_b"]
    return node_new, enewa, enewb


def kernel(node_fea, edge_fea, idx1, idx2, idx3, params):
    nf = params["v_emb"][node_fea]
    we, be = params["e_emb"]["w"], params["e_emb"]["b"]
    efa = edge_fea[:N_HALF] @ we + be
    efb = edge_fea[N_HALF:] @ we + be
    ixs = (idx1[:N_HALF], idx1[N_HALF:], idx2[:N_HALF], idx2[N_HALF:])
    for layer in params["attns"]:
        nf, efa, efb = _attn_layer(layer, nf, ixs, efa, efb)

    ones = jnp.ones((176, 128), jnp.float32)
    pa = _sum_count_half(efa, ones, ixs[0])
    pb = _sum_count_half(efb, ones, ixs[1])
    pooled = pa + pb
    cnt = pooled[_ANR:_ANR + N_NODES, 0]
    vi_e_bar = pooled[:N_NODES] / jnp.maximum(cnt, 1.0)[:, None]
    crys = jnp.concatenate([vi_e_bar, nf], axis=1)
    cnt3 = jax.ops.segment_sum(jnp.ones((N_NODES,), jnp.float32), idx3,
                               num_segments=N_GRAPHS)
    crys = jax.ops.segment_sum(crys, idx3, num_segments=N_GRAPHS)
    crys = crys / jnp.maximum(cnt3, 1.0)[:, None]
    h = _leaky(crys @ params["conv_to_fc"]["w"] + params["conv_to_fc"]["b"])
    for fc in params["fcs"]:
        h = _leaky(h @ fc["w"] + fc["b"])
    return h @ params["fc_out"]["w"] + params["fc_out"]["b"]


# speedup vs baseline: 1.1217x; 1.0001x over previous
import jax
import jax.numpy as jnp
from jax import lax
from jax.experimental import pallas as pl
from jax.experimental.pallas import tpu as pltpu
from jax.experimental.pallas import tpu_sc as plsc

N_NODES = 10000
N_EDGES = 320000
N_HALF = N_EDGES // 2
N_GRAPHS = 256
E_BLK = 3200

_NC = 2
_NS = 16
_NW = _NC * _NS
_ANR = 10240
_APS = _ANR // _NS


def _leaky(x):
    return jnp.where(x >= 0, x, 0.2 * x)


def _mesh():
    return plsc.VectorSubcoreMesh(core_axis_name="c", subcore_axis_name="s")


def _make_gather(n_rows, gc, nb=4):
    epw = n_rows // _NW
    nch = epw // gc
    ngrp = (nch - 1) // nb

    def body(nf_hbm, idx1_hbm, idx2_hbm, n1_hbm, n2_hbm,
             idx1_v, idx2_v, b0, b1, b2, b3,
             g0, g1, g2, g3, w0, w1, w2, w3):
        bufs = (b0, b1, b2, b3)
        gs = (g0, g1, g2, g3)
        ws = (w0, w1, w2, w3)
        cid = lax.axis_index("c")
        sid = lax.axis_index("s")
        wid = sid * _NC + cid
        base = wid * epw
        pltpu.sync_copy(idx1_hbm.at[pl.ds(base, epw)], idx1_v)
        pltpu.sync_copy(idx2_hbm.at[pl.ds(base, epw)], idx2_v)

        def phase(idx_v, out_hbm):
            def start_g(b, c):
                pltpu.async_copy(
                    nf_hbm.at[idx_v.at[pl.ds(c * gc, gc)]], bufs[b], gs[b])

            def wait_g(b, c):
                pltpu.make_async_copy(
                    nf_hbm.at[idx_v.at[pl.ds(c * gc, gc)]],
                    bufs[b], gs[b]).wait()

            def start_w(b, c):
                pltpu.async_copy(
                    bufs[b], out_hbm.at[pl.ds(base + c * gc, gc)], ws[b])

            def wait_w(b, c):
                pltpu.make_async_copy(
                    bufs[b], out_hbm.at[pl.ds(base + c * gc, gc)],
                    ws[b]).wait()

            start_g(0, 0)
            start_g(1, 1)

            def group(i, carry):
                c0 = i * nb
                for j in range(nb):
                    c = c0 + j
                    bn = (j + 2) % nb

                    @pl.when(c + 2 < nch)
                    def _pf():
                        @pl.when(c >= 2)
                        def _dr():
                            wait_w(bn, c - 2)
                        start_g(bn, c + 2)
                    wait_g(j, c)
                    start_w(j, c)
                return carry

            lax.fori_loop(0, ngrp, group, 0)
            for c in range(ngrp * nb, nch):
                wait_g(c % nb, c)
                start_w(c % nb, c)
            for c in range(nch - nb, nch):
                wait_w(c % nb, c)

        phase(idx1_v, n1_hbm)
        phase(idx2_v, n2_hbm)

    def call(nf, idx1, idx2):
        k = pl.kernel(
            body,
            out_type=[jax.ShapeDtypeStruct((n_rows, 128), jnp.float32)] * 2,
            mesh=_mesh(),
            scratch_types=[
                pltpu.VMEM((epw,), jnp.int32),
                pltpu.VMEM((epw,), jnp.int32),
            ] + [pltpu.VMEM((gc, 128), jnp.float32)] * 4
            + [pltpu.SemaphoreType.DMA] * 8,
        )
        return k(nf, idx1, idx2)

    return call


_gather_half = _make_gather(N_HALF, 200)


_PAD_ROW = 10200


def _make_scatter(n_rows, gcs, count_mode=False):
    eps = n_rows // _NS
    nch = eps // gcs
    npair = nch // 2
    tail = eps - nch * gcs
    assert tail % 8 == 0 and (gcs - tail) % 16 == 0 or tail == 0

    def fetch(v_hbm, idx_hbm, off, idx_b, row_b, sem):
        pltpu.async_copy(idx_hbm.at[pl.ds(off, gcs)], idx_b, sem)
        pltpu.async_copy(v_hbm.at[pl.ds(off, gcs)], row_b, sem)

    def await_fetch(v_hbm, idx_hbm, off, idx_b, row_b, sem):
        pltpu.make_async_copy(idx_hbm.at[pl.ds(off, gcs)], idx_b, sem).wait()
        pltpu.make_async_copy(v_hbm.at[pl.ds(off, gcs)], row_b, sem).wait()

    def value_loop(v_hbm, idx_hbm, sid, acc,
                   idx_a, idx_b, row_a, row_b, sem_a, sem_b):
        base = sid * eps
        fetch(v_hbm, idx_hbm, base, idx_a, row_a, sem_a)

        def pair(i, carry):
            off_a = base + 2 * i * gcs
            off_b = off_a + gcs
            fetch(v_hbm, idx_hbm, off_b, idx_b, row_b, sem_b)
            await_fetch(v_hbm, idx_hbm, off_a, idx_a, row_a, sem_a)
            pltpu.sync_copy(row_a, acc.at[idx_a], add=True)

            @pl.when(i < npair - 1)
            def _next_a():
                fetch(v_hbm, idx_hbm, off_b + gcs, idx_a, row_a, sem_a)
            await_fetch(v_hbm, idx_hbm, off_b, idx_b, row_b, sem_b)
            pltpu.sync_copy(row_b, acc.at[idx_b], add=True)
            return carry

        lax.fori_loop(0, npair, pair, 0)
        if nch % 2 == 1:
            off_l = base + (nch - 1) * gcs
            fetch(v_hbm, idx_hbm, off_l, idx_a, row_a, sem_a)
            await_fetch(v_hbm, idx_hbm, off_l, idx_a, row_a, sem_a)
            pltpu.sync_copy(row_a, acc.at[idx_a], add=True)
        if tail:
            off_t = base + nch * gcs
            pltpu.sync_copy(idx_hbm.at[pl.ds(off_t, tail)],
                            idx_a.at[pl.ds(0, tail)])
            for k in range((gcs - tail) // 16):
                idx_a[pl.ds(tail + 16 * k, 16)] = jnp.full(
                    (16,), _PAD_ROW, jnp.int32)
            pltpu.sync_copy(v_hbm.at[pl.ds(off_t, tail)],
                            row_a.at[pl.ds(0, tail)])
            pltpu.sync_copy(row_a, acc.at[idx_a], add=True)

    def count_loop(idx_hbm, sid, acc, idx_a, idx_b, row_a, sem_a, sem_b):
        base = sid * eps
        pltpu.async_copy(idx_hbm.at[pl.ds(base, gcs)], idx_a, sem_a)

        def chunk(c, carry):
            off = base + c * gcs
            noff = off + gcs

            @pl.when((c % 2 == 0) & (c < nch - 1))
            def _pf_b():
                pltpu.async_copy(idx_hbm.at[pl.ds(noff, gcs)], idx_b, sem_b)

            @pl.when(c % 2 == 0)
            def _even():
                pltpu.make_async_copy(
                    idx_hbm.at[pl.ds(off, gcs)], idx_a, sem_a).wait()
                pltpu.sync_copy(row_a, acc.at[idx_a], add=True)

            @pl.when((c % 2 == 1) & (c < nch - 1))
            def _pf_a():
                pltpu.async_copy(idx_hbm.at[pl.ds(noff, gcs)], idx_a, sem_a)

            @pl.when(c % 2 == 1)
            def _odd():
                pltpu.make_async_copy(
                    idx_hbm.at[pl.ds(off, gcs)], idx_b, sem_b).wait()
                pltpu.sync_copy(row_a, acc.at[idx_b], add=True)
            return carry
        lax.fori_loop(0, nch, chunk, 0)
        if tail:
            off_t = base + nch * gcs
            pltpu.sync_copy(idx_hbm.at[pl.ds(off_t, tail)],
                            idx_a.at[pl.ds(0, tail)])
            for k in range((gcs - tail) // 16):
                idx_a[pl.ds(tail + 16 * k, 16)] = jnp.full(
                    (16,), _PAD_ROW, jnp.int32)
            pltpu.sync_copy(row_a, acc.at[idx_a], add=True)

    def body(v1_hbm, v2_hbm, idx_hbm, zz_hbm, out_hbm,
             idx_a, idx_b, row_a, row_b, sem_a, sem_b, acc):
        cid = lax.axis_index("c")
        sid = lax.axis_index("s")
        pltpu.sync_copy(zz_hbm, acc.at[pl.ds(sid * _APS, _APS)])
        plsc.subcore_barrier()

        @pl.when(cid == 0)
        def _core0():
            value_loop(v1_hbm, idx_hbm, sid, acc,
                       idx_a, idx_b, row_a, row_b, sem_a, sem_b)

        @pl.when(cid == 1)
        def _core1():
            if count_mode:
                pltpu.sync_copy(v2_hbm, row_a)
                count_loop(idx_hbm, sid, acc, idx_a, idx_b, row_a,
                           sem_a, sem_b)
            else:
                value_loop(v2_hbm, idx_hbm, sid, acc,
                           idx_a, idx_b, row_a, row_b, sem_a, sem_b)

        plsc.subcore_barrier()
        pltpu.sync_copy(acc.at[pl.ds(sid * _APS, _APS)],
                        out_hbm.at[pl.ds(cid * _ANR + sid * _APS, _APS)])

    def call(v1, v2, idx1):
        zz = jnp.zeros((_APS, 128), jnp.float32)
        k = pl.kernel(
            body,
            out_type=jax.ShapeDtypeStruct((2 * _ANR, 128), jnp.float32),
            mesh=_mesh(),
            scratch_types=[
                pltpu.VMEM((gcs,), jnp.int32),
                pltpu.VMEM((gcs,), jnp.int32),
                pltpu.VMEM((gcs, 128), jnp.float32),
                pltpu.VMEM((gcs, 128), jnp.float32),
                pltpu.SemaphoreType.DMA,
                pltpu.SemaphoreType.DMA,
                pltpu.VMEM_SHARED((_ANR, 128), jnp.float32),
            ],
        )
        return k(v1, v2, idx1, zz)

    return call


_scatter_half = _make_scatter(N_HALF, 176)
_sum_count_half = _make_scatter(N_HALF, 176, count_mode=True)


def _p1_body(n1_ref, n2_ref, ef_ref, wa_ref, wb_ref, wc_ref, b0_ref,
             w1_ref, b1_ref, w2_ref, b2_ref, h3_ref, stats_ref):
    i = pl.program_id(0)
    h = (jnp.dot(n1_ref[...], wa_ref[...], preferred_element_type=jnp.float32, precision=lax.Precision.DEFAULT)
         + jnp.dot(n2_ref[...], wb_ref[...], preferred_element_type=jnp.float32, precision=lax.Precision.DEFAULT)
         + jnp.dot(ef_ref[...], wc_ref[...], preferred_element_type=jnp.float32, precision=lax.Precision.DEFAULT)
         + b0_ref[...])
    h = _leaky(h)
    h = _leaky(jnp.dot(h, w1_ref[...], preferred_element_type=jnp.float32, precision=lax.Precision.DEFAULT)
               + b1_ref[...])
    h3 = jnp.dot(h, w2_ref[...], preferred_element_type=jnp.float32, precision=lax.Precision.DEFAULT) + b2_ref[...]
    h3_ref[...] = h3

    @pl.when(i == 0)
    def _init():
        stats_ref[...] = jnp.zeros_like(stats_ref)

    s1 = jnp.sum(h3, axis=0, keepdims=True)
    s2 = jnp.sum(h3 * h3, axis=0, keepdims=True)
    stats_ref[...] += jnp.concatenate(
        [s1, s2, jnp.zeros((6, h3.shape[1]), jnp.float32)], axis=0)


def _edge_pass1(n1, n2, ef, wa, wb, wc, b0, w1, b1, w2, b2):
    nrows = n1.shape[0]
    eb = lambda i: (i, 0)
    fb = lambda i: (0, 0)
    espec = pl.BlockSpec((E_BLK, 128), eb)
    wspec = pl.BlockSpec((128, 128), fb)
    vspec = pl.BlockSpec((1, 128), fb)
    return pl.pallas_call(
        _p1_body,
        grid=(nrows // E_BLK,),
        in_specs=[espec, espec, espec, wspec, wspec, wspec, vspec,
                  wspec, vspec, wspec, vspec],
        out_specs=[espec, pl.BlockSpec((8, 128), fb)],
        out_shape=[
            jax.ShapeDtypeStruct((nrows, 128), jnp.float32),
            jax.ShapeDtypeStruct((8, 128), jnp.float32),
        ],
    )(n1, n2, ef, wa, wb, wc, b0, w1, b1, w2, b2)


def _p2_body(h3_ref, n1_ref, n2_ref, ef_ref, scale_ref, shift_ref,
             wsa_ref, wsb_ref, wsc_ref, bs0_ref, ws1_ref, bs1_ref,
             wma_ref, wmb_ref, wmc_ref, bm0_ref, wm1_ref, bm1_ref,
             enew_ref, v1_ref, v2_ref):
    ek = h3_ref[...] * scale_ref[...] + shift_ref[...]
    enew_ref[...] = ef_ref[...] + ek
    n1 = n1_ref[...]
    n2 = n2_ref[...]
    sh = _leaky(
        jnp.dot(n1, wsa_ref[...], preferred_element_type=jnp.float32, precision=lax.Precision.DEFAULT)
        + jnp.dot(n2, wsb_ref[...], preferred_element_type=jnp.float32, precision=lax.Precision.DEFAULT)
        + jnp.dot(ek, wsc_ref[...], preferred_element_type=jnp.float32, precision=lax.Precision.DEFAULT)
        + bs0_ref[...])
    sij = jnp.dot(sh, ws1_ref[...], preferred_element_type=jnp.float32, precision=lax.Precision.DEFAULT) + bs1_ref[...]
    w = jnp.exp(jnp.minimum(sij, 60.0))
    mh = _leaky(
        jnp.dot(n1, wma_ref[...], preferred_element_type=jnp.float32, precision=lax.Precision.DEFAULT)
        + jnp.dot(n2, wmb_ref[...], preferred_element_type=jnp.float32, precision=lax.Precision.DEFAULT)
        + jnp.dot(ek, wmc_ref[...], preferred_element_type=jnp.float32, precision=lax.Precision.DEFAULT)
        + bm0_ref[...])
    mij = jnp.dot(mh, wm1_ref[...],
                  preferred_element_type=jnp.float32, precision=lax.Precision.DEFAULT) + bm1_ref[...]
    v1_ref[...] = w * mij
    v2_ref[...] = w


def _edge_pass2(h3, n1, n2, ef, scale, shift, wsa, wsb, wsc, bs0, ws1, bs1,
                wma, wmb, wmc, bm0, wm1, bm1):
    nrows = n1.shape[0]
    eb = lambda i: (i, 0)
    fb = lambda i: (0, 0)
    espec = pl.BlockSpec((E_BLK, 128), eb)
    wspec = pl.BlockSpec((128, 128), fb)
    vspec = pl.BlockSpec((1, 128), fb)
    return pl.pallas_call(
        _p2_body,
        grid=(nrows // E_BLK,),
        in_specs=[espec, espec, espec, espec, vspec, vspec,
                  wspec, wspec, wspec, vspec, wspec, vspec,
                  wspec, wspec, wspec, vspec, wspec, vspec],
        out_specs=[espec, espec, espec],
        out_shape=[jax.ShapeDtypeStruct((nrows, 128), jnp.float32)] * 3,
    )(h3, n1, n2, ef, scale, shift, wsa, wsb, wsc, bs0, ws1, bs1,
      wma, wmb, wmc, bm0, wm1, bm1)


def _bn_from_stats(stats, n, g, b, eps=1e-5):
    mean = stats[0] / n
    var = stats[1] / n - mean * mean
    scale = g / jnp.sqrt(var + eps)
    shift = b - mean * scale
    return scale[None, :], shift[None, :]


def _attn_layer(layer, nf, ixs, efa, efb):
    i1a, i1b, i2a, i2b = ixs
    w_phi0 = layer["phi_e"][0]["w"]
    w_a0 = layer["fcnna"][0]["w"]
    w_m0 = layer["fcnnm"][0]["w"]

    p1w = (w_phi0[:128], w_phi0[128:256], w_phi0[256:384],
           layer["phi_e"][0]["b"][None, :],
           layer["phi_e"][1]["w"], layer["phi_e"][1]["b"][None, :],
           layer["phi_e"][2]["w"], layer["phi_e"][2]["b"][None, :])
    p2w = (w_a0[:128], w_a0[128:256], w_a0[256:384],
           layer["fcnna"][0]["b"][None, :],
           layer["fcnna"][1]["w"], layer["fcnna"][1]["b"][None, :],
           w_m0[:128], w_m0[128:256], w_m0[256:384],
           layer["fcnnm"][0]["b"][None, :],
           layer["fcnnm"][1]["w"], layer["fcnnm"][1]["b"][None, :])

    n1a, n2a = _gather_half(nf, i1a, i2a)
    n1b, n2b = _gather_half(nf, i1b, i2b)

    h3a, sta = _edge_pass1(n1a, n2a, efa, *p1w)
    h3b, stb = _edge_pass1(n1b, n2b, efb, *p1w)

    scale, shift = _bn_from_stats(sta + stb, float(N_EDGES),
                                  layer["bn1_g"], layer["bn1_b"])

    enewa, v1a, v2a = _edge_pass2(h3a, n1a, n2a, efa, scale, shift, *p2w)
    sega = _scatter_half(v1a, v2a, i1a)
    enewb, v1b, v2b = _edge_pass2(h3b, n1b, n2b, efb, scale, shift, *p2w)
    segb = _scatter_half(v1b, v2b, i1b)

    seg = sega + segb
    msg = seg[:N_NODES] / (seg[_ANR:_ANR + N_NODES] + 1e-16)

    mu = jnp.mean(msg, axis=0)
    var = jnp.var(msg, axis=0)
    nsc = layer["bn2_g"] / jnp.sqrt(var + 1e-5)
    node_new = nf + (msg - mu) * nsc + layer["bn2_b"]
    return node_new, enewa, enewb


def kernel(node_fea, edge_fea, idx1, idx2, idx3, params):
    nf = params["v_emb"][node_fea]
    we, be = params["e_emb"]["w"], params["e_emb"]["b"]
    efa = edge_fea[:N_HALF] @ we + be
    efb = edge_fea[N_HALF:] @ we + be
    ixs = (idx1[:N_HALF], idx1[N_HALF:], idx2[:N_HALF], idx2[N_HALF:])
    for layer in params["attns"]:
        nf, efa, efb = _attn_layer(layer, nf, ixs, efa, efb)

    ones = jnp.ones((176, 128), jnp.float32)
    pa = _sum_count_half(efa, ones, ixs[0])
    pb = _sum_count_half(efb, ones, ixs[1])
    pooled = pa + pb
    cnt = pooled[_ANR:_ANR + N_NODES, 0]
    vi_e_bar = pooled[:N_NODES] / jnp.maximum(cnt, 1.0)[:, None]
    crys = jnp.concatenate([vi_e_bar, nf], axis=1)
    cnt3 = jax.ops.segment_sum(jnp.ones((N_NODES,), jnp.float32), idx3,
                               num_segments=N_GRAPHS)
    crys = jax.ops.segment_sum(crys, idx3, num_segments=N_GRAPHS)
    crys = crys / jnp.maximum(cnt3, 1.0)[:, None]
    h = _leaky(crys @ params["conv_to_fc"]["w"] + params["conv_to_fc"]["b"])
    for fc in params["fcs"]:
        h = _leaky(h @ fc["w"] + fc["b"])
    return h @ params["fc_out"]["w"] + params["fc_out"]["b"]
